# baseline (speedup 1.0000x reference)
import functools

import jax
import jax.numpy as jnp
from jax import lax
from jax.experimental import pallas as pl
from jax.experimental.pallas import tpu as pltpu
from jax.experimental.pallas import tpu_sc as plsc

_BLK = 128


def _build_embed(B, S, V, D):
    N = B * S
    info = plsc.get_sparse_core_info()
    NC = info.num_cores
    NL = info.num_lanes
    NW = NC * info.num_subcores
    n_per_w = N // NW
    n_blocks = n_per_w // _BLK
    w_per_seq = S // n_per_w
    DGRP = 16

    mesh = plsc.VectorSubcoreMesh(core_axis_name="c", subcore_axis_name="s")

    @functools.partial(
        pl.kernel,
        mesh=mesh,
        out_type=jax.ShapeDtypeStruct((N * D,), jnp.float32),
        scratch_types=[
            pltpu.VMEM((n_blocks, _BLK), jnp.int32),
            pltpu.VMEM((D * n_per_w,), jnp.float32),
            pltpu.VMEM((DGRP * n_per_w,), jnp.float32),
            pltpu.SemaphoreType.DMA,
        ],
        compiler_params=pltpu.CompilerParams(use_tc_tiling_on_sc=False),
    )
    def emb(ids_hbm, tabT_hbm, posT_hbm, out_hbm, ids_v, buf_v, pos_v, sem):
        wid = lax.axis_index("s") * NC + lax.axis_index("c")
        b = wid // w_per_seq
        s0 = (wid % w_per_seq) * n_per_w
        pltpu.sync_copy(ids_hbm.at[pl.ds(wid * n_blocks, n_blocks)], ids_v)

        for q in range(D // DGRP):

            @pl.loop(0, n_blocks)
            def _(j, _q=q):
                idx_row = ids_v.at[j]
                copies = [
                    pltpu.async_copy(
                        tabT_hbm.at[pl.ds((_q * DGRP + k) * V, V)].at[idx_row],
                        buf_v.at[pl.ds((_q * DGRP + k) * n_per_w + j * _BLK, _BLK)],
                        sem,
                    )
                    for k in range(DGRP)
                ]
                for c in copies:
                    c.wait()

        for q in range(D // DGRP):
            seeds = [
                pltpu.async_copy(
                    posT_hbm.at[pl.ds((q * DGRP + k) * S + s0, n_per_w)],
                    pos_v.at[pl.ds(k * n_per_w, n_per_w)],
                    sem,
                )
                for k in range(DGRP)
            ]
            for c in seeds:
                c.wait()

            @pl.loop(0, DGRP)
            def _(k, _q=q):
                base = (_q * DGRP) * n_per_w + k * n_per_w
                pbase = k * n_per_w
                for r in range(n_per_w // NL):
                    o = r * NL
                    buf_v[pl.ds(base + o, NL)] = (
                        buf_v[pl.ds(base + o, NL)] + pos_v[pl.ds(pbase + o, NL)]
                    )

        obase = b * (D * S) + s0
        for q in range(D // DGRP):
            outs = [
                pltpu.async_copy(
                    buf_v.at[pl.ds((q * DGRP + k) * n_per_w, n_per_w)],
                    out_hbm.at[pl.ds(obase + (q * DGRP + k) * S, n_per_w)],
                    sem,
                )
                for k in range(DGRP)
            ]
            for c in outs:
                c.wait()

    return emb


def kernel(token_ids, token_table, pos_table):
    B, S = token_ids.shape
    V, D = token_table.shape
    N = B * S
    ids_2d = token_ids.reshape(N // _BLK, _BLK).astype(jnp.int32)
    tabT = token_table.T.reshape(D * V)
    posT = pos_table.T.reshape(D * S)
    emb = _build_embed(B, S, V, D)
    out = emb(ids_2d, tabT, posT)
    return jnp.swapaxes(out.reshape(B, D, S), 1, 2)


# speedup vs baseline: 5.7586x; 5.7586x over previous
import functools

import jax
import jax.numpy as jnp
from jax import lax
from jax.experimental import pallas as pl
from jax.experimental.pallas import tpu as pltpu
from jax.experimental.pallas import tpu_sc as plsc

_BLK = 128
_TBLK = 2048


def _tc_transpose(tabT):
    D, V = tabT.shape
    grid = (V + _TBLK - 1) // _TBLK

    def body(i_ref, o_ref):
        o_ref[...] = i_ref[...].T

    return pl.pallas_call(
        body,
        grid=(grid,),
        in_specs=[pl.BlockSpec((D, _TBLK), lambda j: (0, j))],
        out_specs=pl.BlockSpec((_TBLK, D), lambda j: (j, 0)),
        out_shape=jax.ShapeDtypeStruct((V, D), jnp.float32),
    )(tabT)


def _build_gather(B, S, V, D):
    N = B * S
    info = plsc.get_sparse_core_info()
    NC = info.num_cores
    NW = NC * info.num_subcores
    n_per_w = N // NW
    n_blocks = n_per_w // _BLK

    mesh = plsc.VectorSubcoreMesh(core_axis_name="c", subcore_axis_name="s")

    @functools.partial(
        pl.kernel,
        mesh=mesh,
        out_type=jax.ShapeDtypeStruct((N, D), jnp.float32),
        scratch_types=[
            pltpu.VMEM((n_blocks, _BLK), jnp.int32),
            pltpu.VMEM((n_per_w, D), jnp.float32),
            pltpu.SemaphoreType.DMA,
        ],
        compiler_params=pltpu.CompilerParams(use_tc_tiling_on_sc=False),
    )
    def emb(ids_hbm, tab_hbm, pos_hbm, out_hbm, idx_v, buf_v, sem):
        wid = lax.axis_index("s") * NC + lax.axis_index("c")
        base = wid * n_per_w
        p0 = base % S
        pltpu.sync_copy(ids_hbm.at[pl.ds(wid * n_blocks, n_blocks)], idx_v)
        pltpu.sync_copy(pos_hbm.at[pl.ds(p0, n_per_w)], buf_v)
        copies = []
        for j in range(n_blocks):
            copies.append(
                pltpu.async_copy(
                    tab_hbm.at[idx_v.at[j]],
                    buf_v.at[pl.ds(j * _BLK, _BLK)],
                    sem,
                    add=True,
                )
            )
        for c in copies:
            c.wait()
        pltpu.sync_copy(buf_v, out_hbm.at[pl.ds(base, n_per_w)])

    return emb


def kernel(token_ids, token_table, pos_table):
    B, S = token_ids.shape
    V, D = token_table.shape
    N = B * S
    ids_2d = token_ids.reshape(N // _BLK, _BLK).astype(jnp.int32)
    tab_lin = _tc_transpose(token_table.T)
    emb = _build_gather(B, S, V, D)
    out = emb(ids_2d, tab_lin, pos_table)
    return out.reshape(B, S, D)


# speedup vs baseline: 7.3829x; 1.2821x over previous
import functools

import jax
import jax.numpy as jnp
from jax import lax
from jax.experimental import pallas as pl
from jax.experimental.pallas import tpu as pltpu
from jax.experimental.pallas import tpu_sc as plsc

_BLK = 128
_TBLK = 8192


def _tc_transpose(tabT):
    D, V = tabT.shape
    grid = (V + _TBLK - 1) // _TBLK

    def body(i_ref, o_ref):
        o_ref[...] = i_ref[...].T

    return pl.pallas_call(
        body,
        grid=(grid,),
        in_specs=[pl.BlockSpec((D, _TBLK), lambda j: (0, j))],
        out_specs=pl.BlockSpec((_TBLK, D), lambda j: (j, 0)),
        out_shape=jax.ShapeDtypeStruct((V, D), jnp.float32),
    )(tabT)


def _build_gather(B, S, V, D):
    N = B * S
    info = plsc.get_sparse_core_info()
    NC = info.num_cores
    NW = NC * info.num_subcores
    n_per_w = N // NW
    n_blocks = n_per_w // _BLK

    mesh = plsc.VectorSubcoreMesh(core_axis_name="c", subcore_axis_name="s")

    @functools.partial(
        pl.kernel,
        mesh=mesh,
        out_type=jax.ShapeDtypeStruct((N, D), jnp.float32),
        scratch_types=[
            pltpu.VMEM((n_blocks, _BLK), jnp.int32),
            pltpu.VMEM((n_per_w, D), jnp.float32),
            pltpu.SemaphoreType.DMA,
        ],
        compiler_params=pltpu.CompilerParams(use_tc_tiling_on_sc=False),
    )
    def emb(ids_hbm, tab_hbm, pos_hbm, out_hbm, idx_v, buf_v, sem):
        wid = lax.axis_index("s") * NC + lax.axis_index("c")
        base = wid * n_per_w
        p0 = base % S
        pltpu.sync_copy(ids_hbm.at[pl.ds(wid * n_blocks, n_blocks)], idx_v)
        pltpu.sync_copy(pos_hbm.at[pl.ds(p0, n_per_w)], buf_v)
        copies = []
        for j in range(n_blocks):
            copies.append(
                pltpu.async_copy(
                    tab_hbm.at[idx_v.at[j]],
                    buf_v.at[pl.ds(j * _BLK, _BLK)],
                    sem,
                    add=True,
                )
            )
        for c in copies:
            c.wait()
        pltpu.sync_copy(buf_v, out_hbm.at[pl.ds(base, n_per_w)])

    return emb


def kernel(token_ids, token_table, pos_table):
    B, S = token_ids.shape
    V, D = token_table.shape
    N = B * S
    ids_2d = token_ids.reshape(N // _BLK, _BLK).astype(jnp.int32)
    tab_lin = _tc_transpose(token_table.T)
    emb = _build_gather(B, S, V, D)
    out = emb(ids_2d, tab_lin, pos_table)
    return out.reshape(B, S, D)


# speedup vs baseline: 7.6583x; 1.0373x over previous
import functools

import jax
import jax.numpy as jnp
from jax import lax
from jax.experimental import pallas as pl
from jax.experimental.pallas import tpu as pltpu
from jax.experimental.pallas import tpu_sc as plsc

_BLK = 128
_TBLK = 32768


def _tc_transpose(tabT):
    D, V = tabT.shape
    grid = (V + _TBLK - 1) // _TBLK

    def body(i_ref, o_ref):
        o_ref[...] = i_ref[...].T

    return pl.pallas_call(
        body,
        grid=(grid,),
        in_specs=[pl.BlockSpec((D, _TBLK), lambda j: (0, j))],
        out_specs=pl.BlockSpec((_TBLK, D), lambda j: (j, 0)),
        out_shape=jax.ShapeDtypeStruct((V, D), jnp.float32),
    )(tabT)


def _build_gather(B, S, V, D):
    N = B * S
    info = plsc.get_sparse_core_info()
    NC = info.num_cores
    NW = NC * info.num_subcores
    n_per_w = N // NW
    n_blocks = n_per_w // _BLK

    mesh = plsc.VectorSubcoreMesh(core_axis_name="c", subcore_axis_name="s")

    @functools.partial(
        pl.kernel,
        mesh=mesh,
        out_type=jax.ShapeDtypeStruct((N, D), jnp.float32),
        scratch_types=[
            pltpu.VMEM((n_blocks, _BLK), jnp.int32),
            pltpu.VMEM((n_per_w, D), jnp.float32),
            pltpu.SemaphoreType.DMA,
        ],
        compiler_params=pltpu.CompilerParams(use_tc_tiling_on_sc=False),
    )
    def emb(ids_hbm, tab_hbm, pos_hbm, out_hbm, idx_v, buf_v, sem):
        wid = lax.axis_index("s") * NC + lax.axis_index("c")
        base = wid * n_per_w
        p0 = base % S
        pltpu.sync_copy(ids_hbm.at[pl.ds(wid * n_blocks, n_blocks)], idx_v)
        pltpu.sync_copy(pos_hbm.at[pl.ds(p0, n_per_w)], buf_v)
        copies = []
        for j in range(n_blocks):
            copies.append(
                pltpu.async_copy(
                    tab_hbm.at[idx_v.at[j]],
                    buf_v.at[pl.ds(j * _BLK, _BLK)],
                    sem,
                    add=True,
                )
            )
        for c in copies:
            c.wait()
        pltpu.sync_copy(buf_v, out_hbm.at[pl.ds(base, n_per_w)])

    return emb


def kernel(token_ids, token_table, pos_table):
    B, S = token_ids.shape
    V, D = token_table.shape
    N = B * S
    ids_2d = token_ids.reshape(N // _BLK, _BLK).astype(jnp.int32)
    tab_lin = _tc_transpose(token_table.T)
    emb = _build_gather(B, S, V, D)
    out = emb(ids_2d, tab_lin, pos_table)
    return out.reshape(B, S, D)
